# baseline (speedup 1.0000x reference)
import functools

import jax
import jax.numpy as jnp
from jax import lax
from jax.experimental import pallas as pl
from jax.experimental.pallas import tpu as pltpu
from jax.experimental.pallas import tpu_sc as plsc

D = 128
H = 8
T = 104
BN = 512
BE = 512
CHUNK = 128
F32 = jnp.float32


def _mm(a, b):
    return jnp.dot(a, b, preferred_element_type=F32)


def _ln(y):
    m = jnp.mean(y, axis=-1, keepdims=True)
    yc = y - m
    v = jnp.mean(yc * yc, axis=-1, keepdims=True)
    return yc / jnp.sqrt(v + 1e-5)


def _head_sum_mask():
    return (lax.broadcasted_iota(jnp.int32, (D, H), 0) // 16
            == lax.broadcasted_iota(jnp.int32, (D, H), 1)).astype(F32)


def _head_expand_mask(rows):
    return (lax.broadcasted_iota(jnp.int32, (rows, D), 1) // 16
            == lax.broadcasted_iota(jnp.int32, (rows, D), 0)).astype(F32)


def _prep(h2d, atom128, wq, wk, wv, interpret=False):
    NPl = h2d.shape[0]

    def body(h_ref, a_ref, wq_ref, wk_ref, wv_ref, hv_ref, q_ref, k_ref, v_ref):
        hb = h_ref[...]
        oh = (lax.broadcasted_iota(jnp.int32, (BN, 128), 1) == hb).astype(F32)
        hv = _mm(oh, a_ref[...])
        hv_ref[...] = hv
        q_ref[...] = _mm(hv, wq_ref[...]) * 0.25
        k_ref[...] = _mm(hv, wk_ref[...])
        v_ref[...] = _mm(hv, wv_ref[...])

    full = lambda shape: pl.BlockSpec(shape, lambda i: (0, 0))
    return pl.pallas_call(
        body, grid=(NPl // BN,),
        in_specs=[pl.BlockSpec((BN, 1), lambda i: (i, 0)), full((128, D)),
                  full((D, D)), full((D, D)), full((D, D))],
        out_specs=[pl.BlockSpec((BN, D), lambda i: (i, 0))] * 4,
        out_shape=[jax.ShapeDtypeStruct((NPl, D), F32)] * 4,
        interpret=interpret,
    )(h2d, atom128, wq, wk, wv)


def _edge0(e2d, pp, vs, bond16, we0, woe, we1, we2, wE1, E,
           interpret=False):
    EPl = e2d.shape[0]

    def body(e_ref, p_ref, vs_ref, b_ref, we0_ref, woe_ref,
             we1_ref, we2_ref, wE1_ref, rows_ref, ep1_ref):
        i = pl.program_id(0)
        eb = e_ref[...]
        oh = (lax.broadcasted_iota(jnp.int32, (BE, 16), 1) == eb).astype(F32)
        bond = b_ref[...]
        ee = _mm(oh, bond)
        ep0 = _mm(oh, _mm(bond, we0_ref[...]))
        score = p_ref[...] * ep0
        hs = _mm(score, _head_sum_mask())
        rid = lax.broadcasted_iota(jnp.int32, (BE, 1), 0) + i * BE
        msk = (rid < E).astype(F32)
        s = jnp.exp(jnp.clip(hs, -5.0, 5.0)) * msk
        sexp = _mm(s, _head_expand_mask(H))
        rows_ref[...] = jnp.concatenate(
            [vs_ref[...] * sexp, s, jnp.zeros((BE, 8), F32)], axis=1)
        e1 = _ln(ee + _mm(score, woe_ref[...]))
        hmid = jnp.maximum(_mm(e1, we1_ref[...]), 0.0)
        e2 = _ln(e1 + _mm(hmid, we2_ref[...]))
        ep1_ref[...] = _mm(e2, wE1_ref[...])

    full = lambda shape: pl.BlockSpec(shape, lambda i: (0, 0))
    eb_spec = pl.BlockSpec((BE, D), lambda i: (i, 0))
    return pl.pallas_call(
        body, grid=(EPl // BE,),
        in_specs=[pl.BlockSpec((BE, 1), lambda i: (i, 0)), eb_spec,
                  eb_spec, full((16, D)), full((D, D)), full((D, D)),
                  full((D, 2 * D)), full((2 * D, D)), full((D, D))],
        out_specs=[pl.BlockSpec((BE, 144), lambda i: (i, 0)), eb_spec],
        out_shape=[jax.ShapeDtypeStruct((EPl, 144), F32),
                   jax.ShapeDtypeStruct((EPl, D), F32)],
        interpret=interpret,
    )(e2d, pp, vs, bond16, we0, woe, we1, we2, wE1)


def _edge1(pp, vs, ep1, E, interpret=False):
    EPl = pp.shape[0]

    def body(p_ref, vs_ref, ep_ref, rows_ref):
        i = pl.program_id(0)
        score = p_ref[...] * ep_ref[...]
        hs = _mm(score, _head_sum_mask())
        rid = lax.broadcasted_iota(jnp.int32, (BE, 1), 0) + i * BE
        msk = (rid < E).astype(F32)
        s = jnp.exp(jnp.clip(hs, -5.0, 5.0)) * msk
        sexp = _mm(s, _head_expand_mask(H))
        rows_ref[...] = jnp.concatenate(
            [vs_ref[...] * sexp, s, jnp.zeros((BE, 8), F32)], axis=1)

    eb_spec = pl.BlockSpec((BE, D), lambda i: (i, 0))
    return pl.pallas_call(
        body, grid=(EPl // BE,),
        in_specs=[eb_spec, eb_spec, eb_spec],
        out_specs=pl.BlockSpec((BE, 144), lambda i: (i, 0)),
        out_shape=jax.ShapeDtypeStruct((EPl, 144), F32),
        interpret=interpret,
    )(pp, vs, ep1)


def _node_block(a_ref, hprev_ref, wo_ref, wh1_ref, wh2_ref):
    a = a_ref[0] + a_ref[1]
    num = a[:, :D]
    den8 = a[:, D:D + 8]
    h_att = num / (_mm(den8, _head_expand_mask(8)) + 1e-6)
    h1 = _ln(hprev_ref[...] + _mm(h_att, wo_ref[...]))
    return _ln(h1 + _mm(jnp.maximum(_mm(h1, wh1_ref[...]), 0.0), wh2_ref[...]))


def _mid(acc3, hv0, wo, wh1, wh2, wq, wk, wv, interpret=False):
    NPl = hv0.shape[0]

    def body(a_ref, hv_ref, wo_ref, wh1_ref, wh2_ref,
             wq_ref, wk_ref, wv_ref, h2_ref, q_ref, k_ref, v_ref):
        h2 = _node_block(a_ref, hv_ref, wo_ref, wh1_ref, wh2_ref)
        h2_ref[...] = h2
        q_ref[...] = _mm(h2, wq_ref[...]) * 0.25
        k_ref[...] = _mm(h2, wk_ref[...])
        v_ref[...] = _mm(h2, wv_ref[...])

    full = lambda shape: pl.BlockSpec(shape, lambda i: (0, 0))
    nb = pl.BlockSpec((BN, D), lambda i: (i, 0))
    return pl.pallas_call(
        body, grid=(NPl // BN,),
        in_specs=[pl.BlockSpec((2, BN, 144), lambda i: (0, i, 0)),
                  nb, full((D, D)), full((D, 2 * D)), full((2 * D, D)),
                  full((D, D)), full((D, D)), full((D, D))],
        out_specs=[nb] * 4,
        out_shape=[jax.ShapeDtypeStruct((NPl, D), F32)] * 4,
        interpret=interpret,
    )(acc3, hv0, wo, wh1, wh2, wq, wk, wv)


def _fin(acc3, h2prev, wo, wh1, wh2, fnh, fnhb, interpret=False):
    NPl = h2prev.shape[0]

    def body(a_ref, hp_ref, wo_ref, wh1_ref, wh2_ref,
             fnh_ref, fb_ref, hsc_ref):
        h2 = _node_block(a_ref, hp_ref, wo_ref, wh1_ref, wh2_ref)
        hsc_ref[...] = _mm(h2, fnh_ref[...]) + fb_ref[0:1, 0:1]

    full = lambda shape: pl.BlockSpec(shape, lambda i: (0, 0))
    nb = pl.BlockSpec((BN, D), lambda i: (i, 0))
    return pl.pallas_call(
        body, grid=(NPl // BN,),
        in_specs=[pl.BlockSpec((2, BN, 144), lambda i: (0, i, 0)),
                  nb, full((D, D)), full((D, 2 * D)), full((2 * D, D)),
                  full((D, 1)), full((1, 1))],
        out_specs=pl.BlockSpec((BN, 1), lambda i: (i, 0)),
        out_shape=jax.ShapeDtypeStruct((NPl, 1), F32),
        interpret=interpret,
    )(acc3, h2prev, wo, wh1, wh2, fnh, fnhb)


def _hist(sb8, sdp, hsc, interpret=False):
    NPl = sdp.shape[0]

    def body(sb_ref, sd_ref, hs_ref, out_ref):
        i = pl.program_id(0)

        @pl.when(i == 0)
        def _init():
            out_ref[...] = jnp.zeros((T, 1), F32)

        start = sb_ref[0:1, 0:1]
        end = sb_ref[1:2, 0:1]
        b = (sd_ref[...] - 1) // 7 + 1
        valid = ((b + 3) >= start) & ((b + 3) <= end)
        hv = hs_ref[...]
        acc = jnp.zeros((T, 1), F32)
        for dtap in range(4):
            pos = b + dtap - start
            m = valid & (pos >= 0) & (pos < T)
            contrib = hv * ((1.0 - 0.25 * dtap) * m.astype(F32))
            posc = jnp.clip(pos, 0, T - 1)
            oh = (lax.broadcasted_iota(jnp.int32, (BN, T), 1) == posc).astype(F32)
            acc = acc + lax.dot_general(oh, contrib, (((0,), (0,)), ((), ())),
                                        preferred_element_type=F32)
        out_ref[...] = out_ref[...] + acc

    full = lambda shape: pl.BlockSpec(shape, lambda i: (0, 0))
    return pl.pallas_call(
        body, grid=(NPl // BN,),
        in_specs=[full((8, 1)), pl.BlockSpec((BN, 1), lambda i: (i, 0)),
                  pl.BlockSpec((BN, 1), lambda i: (i, 0))],
        out_specs=pl.BlockSpec((T, 1), lambda i: (0, 0)),
        out_shape=jax.ShapeDtypeStruct((T, 1), F32),
        interpret=interpret,
    )(sb8, sdp, hsc)


def _mlp(ei, x104, L0, fnxh8, fnxb, w1, b1row, w2, b2, interpret=False):
    def body(ei_ref, x_ref, fw_ref, fb_ref, w1_ref, b1_ref, w2_ref, b2_ref, o_ref):
        eiv = ei_ref[...]
        enh = (x_ref[...] * fw_ref[0:1, 0:1] + eiv * fw_ref[1:2, 0:1]
               + fb_ref[0:1, 0:1])
        idx = lax.broadcasted_iota(jnp.int32, (T, 1), 0)
        final = jnp.where(idx < L0, enh, eiv)
        hid = jnp.maximum(final * w1_ref[...] + b1_ref[...], 0.0)
        o_ref[...] = _mm(hid, w2_ref[...]) + b2_ref[0:1, 0:1]

    full = lambda shape: pl.BlockSpec(shape, lambda i: (0, 0))
    return pl.pallas_call(
        body, grid=(1,),
        in_specs=[full((T, 1)), full((T, 1)), full((8, 1)), full((1, 1)),
                  full((1, 64)), full((1, 64)), full((64, 1)), full((1, 1))],
        out_specs=full((T, 1)),
        out_shape=jax.ShapeDtypeStruct((T, 1), F32),
        interpret=interpret,
    )(ei, x104, fnxh8, fnxb, w1, b1row, w2, b2)


def _sc_gather_pv(k_t, q_t, v_t, src2d, dst2d):
    info = plsc.get_sparse_core_info()
    NC, NS = info.num_cores, info.num_subcores
    NW = NC * NS
    NCH = src2d.shape[0]
    CH = NCH // NW
    G = CH // 2
    Dl = k_t.shape[1]
    EPl = NCH * CHUNK
    mesh = plsc.VectorSubcoreMesh(core_axis_name="c", subcore_axis_name="s")

    @functools.partial(
        pl.kernel,
        out_type=[jax.ShapeDtypeStruct((EPl, Dl), F32)] * 2,
        mesh=mesh,
        scratch_types=[pltpu.VMEM((CH, CHUNK), jnp.int32),
                       pltpu.VMEM((CH, CHUNK), jnp.int32)]
                      + [pltpu.VMEM((CHUNK, Dl), F32)] * 6
                      + [pltpu.SemaphoreType.DMA] * 4,
    )
    def _k(kt, qt, vt, sp, dp, p_o, vs_o, isrc, idst,
           bk0, bq0, bv0, bk1, bq1, bv1, gsem0, gsem1, wsem0, wsem1):
        wid = lax.axis_index("s") * NC + lax.axis_index("c")
        cbase = wid * CH
        pltpu.sync_copy(sp.at[pl.ds(cbase, CH)], isrc)
        pltpu.sync_copy(dp.at[pl.ds(cbase, CH)], idst)
        bufs = ((bk0, bq0, bv0, gsem0, wsem0), (bk1, bq1, bv1, gsem1, wsem1))

        def issue(p, j):
            bk, bq, bv, gsem, _ = bufs[p]
            pltpu.async_copy(kt.at[isrc.at[j]], bk, gsem)
            pltpu.async_copy(qt.at[idst.at[j]], bq, gsem)
            pltpu.async_copy(vt.at[isrc.at[j]], bv, gsem)

        def finish(p, j):
            bk, bq, bv, gsem, wsem = bufs[p]
            for _ in range(3):
                pltpu.make_async_copy(kt.at[isrc.at[0]], bk, gsem).wait()

            def rowfn(r, c):
                for cc in range(Dl // 16):
                    sl = pl.ds(cc * 16, 16)
                    bk[r, sl] = bk[r, sl] * bq[r, sl]
                return c

            lax.fori_loop(0, CHUNK, rowfn, 0)
            base = (cbase + j) * CHUNK
            pltpu.async_copy(bk, p_o.at[pl.ds(base, CHUNK)], wsem)
            pltpu.async_copy(bv, vs_o.at[pl.ds(base, CHUNK)], wsem)

        def drain_wb(p):
            bk, _, _, _, wsem = bufs[p]
            for _ in range(2):
                pltpu.make_async_copy(bk, p_o.at[pl.ds(0, CHUNK)], wsem).wait()

        issue(0, 0)

        def body(g, c):
            jb = 2 * g + 1

            @pl.when(g > 0)
            def _():
                drain_wb(1)

            issue(1, jb)
            finish(0, 2 * g)

            @pl.when(g < G - 1)
            def _():
                drain_wb(0)
                issue(0, 2 * g + 2)

            finish(1, jb)
            return c

        lax.fori_loop(0, G, body, 0)
        drain_wb(0)
        drain_wb(1)

    return _k(k_t, q_t, v_t, src2d, dst2d)


def _sc_scatter(rows, dst2d, zer):
    info = plsc.get_sparse_core_info()
    NC, NS = info.num_cores, info.num_subcores
    NW = NC * NS
    SCH = dst2d.shape[1]
    NCH = dst2d.shape[0]
    CH = NCH // NW
    G = CH // 2
    NPl = zer.shape[0]
    RP = NPl // NS
    mesh = plsc.VectorSubcoreMesh(core_axis_name="c", subcore_axis_name="s")

    @functools.partial(
        pl.kernel,
        out_type=jax.ShapeDtypeStruct((NC * NPl, 144), F32),
        mesh=mesh,
        compiler_params=pltpu.CompilerParams(use_tc_tiling_on_sc=False),
        scratch_types=[pltpu.VMEM((CH, SCH), jnp.int32),
                       pltpu.VMEM((SCH, 144), F32),
                       pltpu.VMEM((SCH, 144), F32),
                       pltpu.VMEM_SHARED((NPl, 144), F32)]
                      + [pltpu.SemaphoreType.DMA] * 4,
    )
    def _k(rw, dp, z, out, idxb, rb0, rb1, acc, fsem0, fsem1, ssem0, ssem1):
        cid = lax.axis_index("c")
        sid = lax.axis_index("s")
        wid = sid * NC + cid
        cbase = wid * CH
        pltpu.sync_copy(dp.at[pl.ds(cbase, CH)], idxb)
        pltpu.sync_copy(z.at[pl.ds(sid * RP, RP)], acc.at[pl.ds(sid * RP, RP)])
        plsc.subcore_barrier()
        bufs = ((rb0, fsem0, ssem0), (rb1, fsem1, ssem1))

        def fetch(p, j):
            rb, fsem, _ = bufs[p]
            pltpu.async_copy(rw.at[pl.ds((cbase + j) * SCH, SCH)], rb, fsem)

        def drain_fetch(p):
            rb, fsem, _ = bufs[p]
            pltpu.make_async_copy(rw.at[pl.ds(0, SCH)], rb, fsem).wait()

        def scat(p, j):
            rb, _, ssem = bufs[p]
            pltpu.async_copy(rb, acc.at[idxb.at[j]], ssem, add=True)

        def drain_scat(p):
            rb, _, ssem = bufs[p]
            pltpu.make_async_copy(rb, acc.at[idxb.at[0]], ssem).wait()

        fetch(0, 0)

        def body(g, c):
            jb = 2 * g + 1

            @pl.when(g > 0)
            def _():
                drain_scat(1)

            fetch(1, jb)
            drain_fetch(0)
            scat(0, 2 * g)

            @pl.when(g < G - 1)
            def _():
                drain_scat(0)
                fetch(0, 2 * g + 2)

            drain_fetch(1)
            scat(1, jb)
            return c

        lax.fori_loop(0, G, body, 0)
        drain_scat(0)
        drain_scat(1)
        plsc.subcore_barrier()
        pltpu.sync_copy(acc.at[pl.ds(sid * RP, RP)],
                        out.at[pl.ds(cid * NPl + sid * RP, RP)])

    return _k(rows, dst2d, zer)


def kernel(x, params, h, e, edge_index, sd, start_bucket, end_bucket):
    p = params
    N = h.shape[0]
    E = e.shape[0]
    NPl = ((N + BN - 1) // BN) * BN
    EPl = ((E + 4095) // 4096) * 4096

    h2d = jnp.pad(h.astype(jnp.int32), (0, NPl - N)).reshape(NPl, 1)
    e2d = jnp.pad(e.astype(jnp.int32), (0, EPl - E)).reshape(EPl, 1)
    srcp = jnp.pad(edge_index[0].astype(jnp.int32), (0, EPl - E))
    dstp = jnp.pad(edge_index[1].astype(jnp.int32), (0, EPl - E))
    sdp = jnp.pad(sd.astype(jnp.int32), (0, NPl - N),
                  constant_values=10 ** 6).reshape(NPl, 1)
    atom128 = jnp.pad(p['atom_emb'], ((0, 128 - p['atom_emb'].shape[0]), (0, 0)))
    bond16 = jnp.pad(p['bond_emb'], ((0, 16 - p['bond_emb'].shape[0]), (0, 0)))
    zer = jnp.zeros((NPl, 144), F32)
    L0 = x.shape[0]
    x104 = jnp.pad(x, ((0, T - L0), (0, 0)))
    sb8 = jnp.zeros((8, 1), jnp.int32)
    sb8 = sb8.at[0, 0].set(jnp.asarray(start_bucket, jnp.int32))
    sb8 = sb8.at[1, 0].set(jnp.asarray(end_bucket, jnp.int32))

    src2d = srcp.reshape(EPl // CHUNK, CHUNK)
    dst2d = dstp.reshape(EPl // CHUNK, CHUNK)
    dst2s = dstp.reshape(EPl // 64, 64)
    hv0, q0, k0, v0 = _prep(h2d, atom128, p['WQ_0'], p['WK_0'], p['WV_0'])
    pp0, vs0 = _sc_gather_pv(k0, q0, v0, src2d, dst2d)
    rows0, ep1 = _edge0(e2d, pp0, vs0, bond16, p['WE_0'], p['WOe_0'],
                        p['We1_0'], p['We2_0'], p['WE_1'], E)
    acc0 = _sc_scatter(rows0, dst2s, zer).reshape(2, NPl, 144)
    h2, q1, k1, v1 = _mid(acc0, hv0, p['WO_0'], p['Wh1_0'], p['Wh2_0'],
                          p['WQ_1'], p['WK_1'], p['WV_1'])
    pp1, vs1 = _sc_gather_pv(k1, q1, v1, src2d, dst2d)
    rows1 = _edge1(pp1, vs1, ep1, E)
    acc1 = _sc_scatter(rows1, dst2s, zer).reshape(2, NPl, 144)
    hsc = _fin(acc1, h2, p['WO_1'], p['Wh1_1'], p['Wh2_1'],
               p['fnh_W'], p['fnh_b'].reshape(1, 1))
    ei = _hist(sb8, sdp, hsc)
    fnxh8 = jnp.pad(p['fnxh_W'].reshape(2, 1), ((0, 6), (0, 0)))
    out = _mlp(ei, x104, L0, fnxh8, p['fnxh_b'].reshape(1, 1),
               p['mlp_W1'], p['mlp_b1'].reshape(1, 64), p['mlp_W2'],
               p['mlp_b2'].reshape(1, 1))
    return out


# speedup vs baseline: 1.1293x; 1.1293x over previous
import functools

import jax
import jax.numpy as jnp
from jax import lax
from jax.experimental import pallas as pl
from jax.experimental.pallas import tpu as pltpu
from jax.experimental.pallas import tpu_sc as plsc

D = 128
H = 8
T = 104
BN = 512
BE = 1024
CHUNK = 128
F32 = jnp.float32


def _mm(a, b):
    return jnp.dot(a, b, preferred_element_type=F32)


def _ln(y):
    m = jnp.mean(y, axis=-1, keepdims=True)
    yc = y - m
    v = jnp.mean(yc * yc, axis=-1, keepdims=True)
    return yc / jnp.sqrt(v + 1e-5)


def _head_sum_mask():
    return (lax.broadcasted_iota(jnp.int32, (D, H), 0) // 16
            == lax.broadcasted_iota(jnp.int32, (D, H), 1)).astype(F32)


def _head_expand_mask(rows):
    return (lax.broadcasted_iota(jnp.int32, (rows, D), 1) // 16
            == lax.broadcasted_iota(jnp.int32, (rows, D), 0)).astype(F32)


def _prep(h2d, atom128, wq, wk, wv, interpret=False):
    NPl = h2d.shape[0]

    def body(h_ref, a_ref, wq_ref, wk_ref, wv_ref, hv_ref, q_ref, k_ref, v_ref):
        hb = h_ref[...]
        oh = (lax.broadcasted_iota(jnp.int32, (BN, 128), 1) == hb).astype(F32)
        hv = _mm(oh, a_ref[...])
        hv_ref[...] = hv
        q_ref[...] = _mm(hv, wq_ref[...]) * 0.25
        k_ref[...] = _mm(hv, wk_ref[...])
        v_ref[...] = _mm(hv, wv_ref[...])

    full = lambda shape: pl.BlockSpec(shape, lambda i: (0, 0))
    return pl.pallas_call(
        body, grid=(NPl // BN,),
        in_specs=[pl.BlockSpec((BN, 1), lambda i: (i, 0)), full((128, D)),
                  full((D, D)), full((D, D)), full((D, D))],
        out_specs=[pl.BlockSpec((BN, D), lambda i: (i, 0))] * 4,
        out_shape=[jax.ShapeDtypeStruct((NPl, D), F32)] * 4,
        interpret=interpret,
    )(h2d, atom128, wq, wk, wv)


def _edge0(e2d, pp, vs, bond16, we0, woe, we1, we2, wE1, E,
           interpret=False):
    EPl = e2d.shape[0]

    def body(e_ref, p_ref, vs_ref, b_ref, we0_ref, woe_ref,
             we1_ref, we2_ref, wE1_ref, rows_ref, ep1_ref):
        i = pl.program_id(0)
        eb = e_ref[...]
        oh = (lax.broadcasted_iota(jnp.int32, (BE, 16), 1) == eb).astype(F32)
        bond = b_ref[...]
        ee = _mm(oh, bond)
        ep0 = _mm(oh, _mm(bond, we0_ref[...]))
        score = p_ref[...] * ep0
        hs = _mm(score, _head_sum_mask())
        rid = lax.broadcasted_iota(jnp.int32, (BE, 1), 0) + i * BE
        msk = (rid < E).astype(F32)
        s = jnp.exp(jnp.clip(hs, -5.0, 5.0)) * msk
        sexp = _mm(s, _head_expand_mask(H))
        rows_ref[...] = jnp.concatenate(
            [vs_ref[...] * sexp, s, jnp.zeros((BE, 8), F32)], axis=1)
        e1 = _ln(ee + _mm(score, woe_ref[...]))
        hmid = jnp.maximum(_mm(e1, we1_ref[...]), 0.0)
        e2 = _ln(e1 + _mm(hmid, we2_ref[...]))
        ep1_ref[...] = _mm(e2, wE1_ref[...])

    full = lambda shape: pl.BlockSpec(shape, lambda i: (0, 0))
    eb_spec = pl.BlockSpec((BE, D), lambda i: (i, 0))
    return pl.pallas_call(
        body, grid=(EPl // BE,),
        in_specs=[pl.BlockSpec((BE, 1), lambda i: (i, 0)), eb_spec,
                  eb_spec, full((16, D)), full((D, D)), full((D, D)),
                  full((D, 2 * D)), full((2 * D, D)), full((D, D))],
        out_specs=[pl.BlockSpec((BE, 144), lambda i: (i, 0)), eb_spec],
        out_shape=[jax.ShapeDtypeStruct((EPl, 144), F32),
                   jax.ShapeDtypeStruct((EPl, D), F32)],
        interpret=interpret,
    )(e2d, pp, vs, bond16, we0, woe, we1, we2, wE1)


def _edge1(pp, vs, ep1, E, interpret=False):
    EPl = pp.shape[0]

    def body(p_ref, vs_ref, ep_ref, rows_ref):
        i = pl.program_id(0)
        score = p_ref[...] * ep_ref[...]
        hs = _mm(score, _head_sum_mask())
        rid = lax.broadcasted_iota(jnp.int32, (BE, 1), 0) + i * BE
        msk = (rid < E).astype(F32)
        s = jnp.exp(jnp.clip(hs, -5.0, 5.0)) * msk
        sexp = _mm(s, _head_expand_mask(H))
        rows_ref[...] = jnp.concatenate(
            [vs_ref[...] * sexp, s, jnp.zeros((BE, 8), F32)], axis=1)

    eb_spec = pl.BlockSpec((BE, D), lambda i: (i, 0))
    return pl.pallas_call(
        body, grid=(EPl // BE,),
        in_specs=[eb_spec, eb_spec, eb_spec],
        out_specs=pl.BlockSpec((BE, 144), lambda i: (i, 0)),
        out_shape=jax.ShapeDtypeStruct((EPl, 144), F32),
        interpret=interpret,
    )(pp, vs, ep1)


def _node_block(a_ref, hprev_ref, wo_ref, wh1_ref, wh2_ref):
    a = a_ref[0] + a_ref[1]
    num = a[:, :D]
    den8 = a[:, D:D + 8]
    h_att = num / (_mm(den8, _head_expand_mask(8)) + 1e-6)
    h1 = _ln(hprev_ref[...] + _mm(h_att, wo_ref[...]))
    return _ln(h1 + _mm(jnp.maximum(_mm(h1, wh1_ref[...]), 0.0), wh2_ref[...]))


def _mid(acc3, hv0, wo, wh1, wh2, wq, wk, wv, interpret=False):
    NPl = hv0.shape[0]

    def body(a_ref, hv_ref, wo_ref, wh1_ref, wh2_ref,
             wq_ref, wk_ref, wv_ref, h2_ref, q_ref, k_ref, v_ref):
        h2 = _node_block(a_ref, hv_ref, wo_ref, wh1_ref, wh2_ref)
        h2_ref[...] = h2
        q_ref[...] = _mm(h2, wq_ref[...]) * 0.25
        k_ref[...] = _mm(h2, wk_ref[...])
        v_ref[...] = _mm(h2, wv_ref[...])

    full = lambda shape: pl.BlockSpec(shape, lambda i: (0, 0))
    nb = pl.BlockSpec((BN, D), lambda i: (i, 0))
    return pl.pallas_call(
        body, grid=(NPl // BN,),
        in_specs=[pl.BlockSpec((2, BN, 144), lambda i: (0, i, 0)),
                  nb, full((D, D)), full((D, 2 * D)), full((2 * D, D)),
                  full((D, D)), full((D, D)), full((D, D))],
        out_specs=[nb] * 4,
        out_shape=[jax.ShapeDtypeStruct((NPl, D), F32)] * 4,
        interpret=interpret,
    )(acc3, hv0, wo, wh1, wh2, wq, wk, wv)


def _fin(acc3, h2prev, wo, wh1, wh2, fnh, fnhb, interpret=False):
    NPl = h2prev.shape[0]

    def body(a_ref, hp_ref, wo_ref, wh1_ref, wh2_ref,
             fnh_ref, fb_ref, hsc_ref):
        h2 = _node_block(a_ref, hp_ref, wo_ref, wh1_ref, wh2_ref)
        hsc_ref[...] = _mm(h2, fnh_ref[...]) + fb_ref[0:1, 0:1]

    full = lambda shape: pl.BlockSpec(shape, lambda i: (0, 0))
    nb = pl.BlockSpec((BN, D), lambda i: (i, 0))
    return pl.pallas_call(
        body, grid=(NPl // BN,),
        in_specs=[pl.BlockSpec((2, BN, 144), lambda i: (0, i, 0)),
                  nb, full((D, D)), full((D, 2 * D)), full((2 * D, D)),
                  full((D, 1)), full((1, 1))],
        out_specs=pl.BlockSpec((BN, 1), lambda i: (i, 0)),
        out_shape=jax.ShapeDtypeStruct((NPl, 1), F32),
        interpret=interpret,
    )(acc3, h2prev, wo, wh1, wh2, fnh, fnhb)


def _hist(sb8, sdp, hsc, interpret=False):
    NPl = sdp.shape[0]

    def body(sb_ref, sd_ref, hs_ref, out_ref):
        i = pl.program_id(0)

        @pl.when(i == 0)
        def _init():
            out_ref[...] = jnp.zeros((T, 1), F32)

        start = sb_ref[0:1, 0:1]
        end = sb_ref[1:2, 0:1]
        b = (sd_ref[...] - 1) // 7 + 1
        valid = ((b + 3) >= start) & ((b + 3) <= end)
        hv = hs_ref[...]
        acc = jnp.zeros((T, 1), F32)
        for dtap in range(4):
            pos = b + dtap - start
            m = valid & (pos >= 0) & (pos < T)
            contrib = hv * ((1.0 - 0.25 * dtap) * m.astype(F32))
            posc = jnp.clip(pos, 0, T - 1)
            oh = (lax.broadcasted_iota(jnp.int32, (BN, T), 1) == posc).astype(F32)
            acc = acc + lax.dot_general(oh, contrib, (((0,), (0,)), ((), ())),
                                        preferred_element_type=F32)
        out_ref[...] = out_ref[...] + acc

    full = lambda shape: pl.BlockSpec(shape, lambda i: (0, 0))
    return pl.pallas_call(
        body, grid=(NPl // BN,),
        in_specs=[full((8, 1)), pl.BlockSpec((BN, 1), lambda i: (i, 0)),
                  pl.BlockSpec((BN, 1), lambda i: (i, 0))],
        out_specs=pl.BlockSpec((T, 1), lambda i: (0, 0)),
        out_shape=jax.ShapeDtypeStruct((T, 1), F32),
        interpret=interpret,
    )(sb8, sdp, hsc)


def _mlp(ei, x104, L0, fnxh8, fnxb, w1, b1row, w2, b2, interpret=False):
    def body(ei_ref, x_ref, fw_ref, fb_ref, w1_ref, b1_ref, w2_ref, b2_ref, o_ref):
        eiv = ei_ref[...]
        enh = (x_ref[...] * fw_ref[0:1, 0:1] + eiv * fw_ref[1:2, 0:1]
               + fb_ref[0:1, 0:1])
        idx = lax.broadcasted_iota(jnp.int32, (T, 1), 0)
        final = jnp.where(idx < L0, enh, eiv)
        hid = jnp.maximum(final * w1_ref[...] + b1_ref[...], 0.0)
        o_ref[...] = _mm(hid, w2_ref[...]) + b2_ref[0:1, 0:1]

    full = lambda shape: pl.BlockSpec(shape, lambda i: (0, 0))
    return pl.pallas_call(
        body, grid=(1,),
        in_specs=[full((T, 1)), full((T, 1)), full((8, 1)), full((1, 1)),
                  full((1, 64)), full((1, 64)), full((64, 1)), full((1, 1))],
        out_specs=full((T, 1)),
        out_shape=jax.ShapeDtypeStruct((T, 1), F32),
        interpret=interpret,
    )(ei, x104, fnxh8, fnxb, w1, b1row, w2, b2)


def _sc_gather_pv(k_t, q_t, v_t, src2d, dst2d):
    info = plsc.get_sparse_core_info()
    NC, NS = info.num_cores, info.num_subcores
    NW = NC * NS
    NCH = src2d.shape[0]
    CH = NCH // NW
    G = CH // 2
    Dl = k_t.shape[1]
    EPl = NCH * CHUNK
    mesh = plsc.VectorSubcoreMesh(core_axis_name="c", subcore_axis_name="s")

    @functools.partial(
        pl.kernel,
        out_type=[jax.ShapeDtypeStruct((EPl, Dl), F32)] * 2,
        mesh=mesh,
        scratch_types=[pltpu.VMEM((CH, CHUNK), jnp.int32),
                       pltpu.VMEM((CH, CHUNK), jnp.int32)]
                      + [pltpu.VMEM((CHUNK, Dl), F32)] * 6
                      + [pltpu.SemaphoreType.DMA] * 4,
    )
    def _k(kt, qt, vt, sp, dp, p_o, vs_o, isrc, idst,
           bk0, bq0, bv0, bk1, bq1, bv1, gsem0, gsem1, wsem0, wsem1):
        wid = lax.axis_index("s") * NC + lax.axis_index("c")
        cbase = wid * CH
        pltpu.sync_copy(sp.at[pl.ds(cbase, CH)], isrc)
        pltpu.sync_copy(dp.at[pl.ds(cbase, CH)], idst)
        bufs = ((bk0, bq0, bv0, gsem0, wsem0), (bk1, bq1, bv1, gsem1, wsem1))

        def issue(p, j):
            bk, bq, bv, gsem, _ = bufs[p]
            pltpu.async_copy(kt.at[isrc.at[j]], bk, gsem)
            pltpu.async_copy(qt.at[idst.at[j]], bq, gsem)
            pltpu.async_copy(vt.at[isrc.at[j]], bv, gsem)

        def finish(p, j):
            bk, bq, bv, gsem, wsem = bufs[p]
            for _ in range(3):
                pltpu.make_async_copy(kt.at[isrc.at[0]], bk, gsem).wait()

            def rowfn(r, c):
                for cc in range(Dl // 16):
                    sl = pl.ds(cc * 16, 16)
                    bk[r, sl] = bk[r, sl] * bq[r, sl]
                return c

            lax.fori_loop(0, CHUNK, rowfn, 0)
            base = (cbase + j) * CHUNK
            pltpu.async_copy(bk, p_o.at[pl.ds(base, CHUNK)], wsem)
            pltpu.async_copy(bv, vs_o.at[pl.ds(base, CHUNK)], wsem)

        def drain_wb(p):
            bk, _, _, _, wsem = bufs[p]
            for _ in range(2):
                pltpu.make_async_copy(bk, p_o.at[pl.ds(0, CHUNK)], wsem).wait()

        issue(0, 0)

        def body(g, c):
            jb = 2 * g + 1

            @pl.when(g > 0)
            def _():
                drain_wb(1)

            issue(1, jb)
            finish(0, 2 * g)

            @pl.when(g < G - 1)
            def _():
                drain_wb(0)
                issue(0, 2 * g + 2)

            finish(1, jb)
            return c

        lax.fori_loop(0, G, body, 0)
        drain_wb(0)
        drain_wb(1)

    return _k(k_t, q_t, v_t, src2d, dst2d)


def _sc_scatter(rows, dst2d, zer):
    info = plsc.get_sparse_core_info()
    NC, NS = info.num_cores, info.num_subcores
    NW = NC * NS
    SCH = dst2d.shape[1]
    NCH = dst2d.shape[0]
    CH = NCH // NW
    G = CH // 2
    NPl = zer.shape[0]
    RP = NPl // NS
    mesh = plsc.VectorSubcoreMesh(core_axis_name="c", subcore_axis_name="s")

    @functools.partial(
        pl.kernel,
        out_type=jax.ShapeDtypeStruct((NC * NPl, 144), F32),
        mesh=mesh,
        compiler_params=pltpu.CompilerParams(use_tc_tiling_on_sc=False),
        scratch_types=[pltpu.VMEM((CH, SCH), jnp.int32),
                       pltpu.VMEM((SCH, 144), F32),
                       pltpu.VMEM((SCH, 144), F32),
                       pltpu.VMEM_SHARED((NPl, 144), F32)]
                      + [pltpu.SemaphoreType.DMA] * 4,
    )
    def _k(rw, dp, z, out, idxb, rb0, rb1, acc, fsem0, fsem1, ssem0, ssem1):
        cid = lax.axis_index("c")
        sid = lax.axis_index("s")
        wid = sid * NC + cid
        cbase = wid * CH
        pltpu.sync_copy(dp.at[pl.ds(cbase, CH)], idxb)
        pltpu.sync_copy(z.at[pl.ds(sid * RP, RP)], acc.at[pl.ds(sid * RP, RP)])
        plsc.subcore_barrier()
        bufs = ((rb0, fsem0, ssem0), (rb1, fsem1, ssem1))

        def fetch(p, j):
            rb, fsem, _ = bufs[p]
            pltpu.async_copy(rw.at[pl.ds((cbase + j) * SCH, SCH)], rb, fsem)

        def drain_fetch(p):
            rb, fsem, _ = bufs[p]
            pltpu.make_async_copy(rw.at[pl.ds(0, SCH)], rb, fsem).wait()

        def scat(p, j):
            rb, _, ssem = bufs[p]
            pltpu.async_copy(rb, acc.at[idxb.at[j]], ssem, add=True)

        def drain_scat(p):
            rb, _, ssem = bufs[p]
            pltpu.make_async_copy(rb, acc.at[idxb.at[0]], ssem).wait()

        fetch(0, 0)

        def body(g, c):
            jb = 2 * g + 1

            @pl.when(g > 0)
            def _():
                drain_scat(1)

            fetch(1, jb)
            drain_fetch(0)
            scat(0, 2 * g)

            @pl.when(g < G - 1)
            def _():
                drain_scat(0)
                fetch(0, 2 * g + 2)

            drain_fetch(1)
            scat(1, jb)
            return c

        lax.fori_loop(0, G, body, 0)
        drain_scat(0)
        drain_scat(1)
        plsc.subcore_barrier()
        pltpu.sync_copy(acc.at[pl.ds(sid * RP, RP)],
                        out.at[pl.ds(cid * NPl + sid * RP, RP)])

    return _k(rows, dst2d, zer)


def kernel(x, params, h, e, edge_index, sd, start_bucket, end_bucket):
    p = params
    N = h.shape[0]
    E = e.shape[0]
    NPl = ((N + BN - 1) // BN) * BN
    EPl = ((E + 4095) // 4096) * 4096

    h2d = jnp.pad(h.astype(jnp.int32), (0, NPl - N)).reshape(NPl, 1)
    e2d = jnp.pad(e.astype(jnp.int32), (0, EPl - E)).reshape(EPl, 1)
    srcp = jnp.pad(edge_index[0].astype(jnp.int32), (0, EPl - E))
    dstp = jnp.pad(edge_index[1].astype(jnp.int32), (0, EPl - E))
    sdp = jnp.pad(sd.astype(jnp.int32), (0, NPl - N),
                  constant_values=10 ** 6).reshape(NPl, 1)
    atom128 = jnp.pad(p['atom_emb'], ((0, 128 - p['atom_emb'].shape[0]), (0, 0)))
    bond16 = jnp.pad(p['bond_emb'], ((0, 16 - p['bond_emb'].shape[0]), (0, 0)))
    zer = jnp.zeros((NPl, 144), F32)
    L0 = x.shape[0]
    x104 = jnp.pad(x, ((0, T - L0), (0, 0)))
    sb8 = jnp.zeros((8, 1), jnp.int32)
    sb8 = sb8.at[0, 0].set(jnp.asarray(start_bucket, jnp.int32))
    sb8 = sb8.at[1, 0].set(jnp.asarray(end_bucket, jnp.int32))

    src2d = srcp.reshape(EPl // CHUNK, CHUNK)
    dst2d = dstp.reshape(EPl // CHUNK, CHUNK)
    dst2s = dstp.reshape(EPl // 64, 64)
    hv0, q0, k0, v0 = _prep(h2d, atom128, p['WQ_0'], p['WK_0'], p['WV_0'])
    pp0, vs0 = _sc_gather_pv(k0, q0, v0, src2d, dst2d)
    rows0, ep1 = _edge0(e2d, pp0, vs0, bond16, p['WE_0'], p['WOe_0'],
                        p['We1_0'], p['We2_0'], p['WE_1'], E)
    acc0 = _sc_scatter(rows0, dst2s, zer).reshape(2, NPl, 144)
    h2, q1, k1, v1 = _mid(acc0, hv0, p['WO_0'], p['Wh1_0'], p['Wh2_0'],
                          p['WQ_1'], p['WK_1'], p['WV_1'])
    pp1, vs1 = _sc_gather_pv(k1, q1, v1, src2d, dst2d)
    rows1 = _edge1(pp1, vs1, ep1, E)
    acc1 = _sc_scatter(rows1, dst2s, zer).reshape(2, NPl, 144)
    hsc = _fin(acc1, h2, p['WO_1'], p['Wh1_1'], p['Wh2_1'],
               p['fnh_W'], p['fnh_b'].reshape(1, 1))
    ei = _hist(sb8, sdp, hsc)
    fnxh8 = jnp.pad(p['fnxh_W'].reshape(2, 1), ((0, 6), (0, 0)))
    out = _mlp(ei, x104, L0, fnxh8, p['fnxh_b'].reshape(1, 1),
               p['mlp_W1'], p['mlp_b1'].reshape(1, 64), p['mlp_W2'],
               p['mlp_b2'].reshape(1, 1))
    return out


# speedup vs baseline: 1.2067x; 1.0685x over previous
import functools

import jax
import jax.numpy as jnp
from jax import lax
from jax.experimental import pallas as pl
from jax.experimental.pallas import tpu as pltpu
from jax.experimental.pallas import tpu_sc as plsc

D = 128
H = 8
T = 104
BN = 512
BE = 2048
CHUNK = 128
F32 = jnp.float32


def _mm(a, b):
    return jnp.dot(a, b, preferred_element_type=F32)


def _ln(y):
    m = jnp.mean(y, axis=-1, keepdims=True)
    yc = y - m
    v = jnp.mean(yc * yc, axis=-1, keepdims=True)
    return yc / jnp.sqrt(v + 1e-5)


def _head_sum_mask():
    return (lax.broadcasted_iota(jnp.int32, (D, H), 0) // 16
            == lax.broadcasted_iota(jnp.int32, (D, H), 1)).astype(F32)


def _head_expand_mask(rows):
    return (lax.broadcasted_iota(jnp.int32, (rows, D), 1) // 16
            == lax.broadcasted_iota(jnp.int32, (rows, D), 0)).astype(F32)


def _prep(h2d, atom128, wq, wk, wv, interpret=False):
    NPl = h2d.shape[0]

    def body(h_ref, a_ref, wq_ref, wk_ref, wv_ref, hv_ref, q_ref, k_ref, v_ref):
        hb = h_ref[...]
        oh = (lax.broadcasted_iota(jnp.int32, (BN, 128), 1) == hb).astype(F32)
        hv = _mm(oh, a_ref[...])
        hv_ref[...] = hv
        q_ref[...] = _mm(hv, wq_ref[...]) * 0.25
        k_ref[...] = _mm(hv, wk_ref[...])
        v_ref[...] = _mm(hv, wv_ref[...])

    full = lambda shape: pl.BlockSpec(shape, lambda i: (0, 0))
    return pl.pallas_call(
        body, grid=(NPl // BN,),
        in_specs=[pl.BlockSpec((BN, 1), lambda i: (i, 0)), full((128, D)),
                  full((D, D)), full((D, D)), full((D, D))],
        out_specs=[pl.BlockSpec((BN, D), lambda i: (i, 0))] * 4,
        out_shape=[jax.ShapeDtypeStruct((NPl, D), F32)] * 4,
        interpret=interpret,
    )(h2d, atom128, wq, wk, wv)


def _edge0(e2d, pp, vs, bond16, we0, woe, we1, we2, wE1, E,
           interpret=False):
    EPl = e2d.shape[0]

    def body(e_ref, p_ref, vs_ref, b_ref, we0_ref, woe_ref,
             we1_ref, we2_ref, wE1_ref, rows_ref, ep1_ref):
        i = pl.program_id(0)
        eb = e_ref[...]
        oh = (lax.broadcasted_iota(jnp.int32, (BE, 16), 1) == eb).astype(F32)
        bond = b_ref[...]
        ee = _mm(oh, bond)
        ep0 = _mm(oh, _mm(bond, we0_ref[...]))
        score = p_ref[...] * ep0
        hs = _mm(score, _head_sum_mask())
        rid = lax.broadcasted_iota(jnp.int32, (BE, 1), 0) + i * BE
        msk = (rid < E).astype(F32)
        s = jnp.exp(jnp.clip(hs, -5.0, 5.0)) * msk
        sexp = _mm(s, _head_expand_mask(H))
        rows_ref[...] = jnp.concatenate(
            [vs_ref[...] * sexp, s, jnp.zeros((BE, 8), F32)], axis=1)
        e1 = _ln(ee + _mm(score, woe_ref[...]))
        hmid = jnp.maximum(_mm(e1, we1_ref[...]), 0.0)
        e2 = _ln(e1 + _mm(hmid, we2_ref[...]))
        ep1_ref[...] = _mm(e2, wE1_ref[...])

    full = lambda shape: pl.BlockSpec(shape, lambda i: (0, 0))
    eb_spec = pl.BlockSpec((BE, D), lambda i: (i, 0))
    return pl.pallas_call(
        body, grid=(EPl // BE,),
        in_specs=[pl.BlockSpec((BE, 1), lambda i: (i, 0)), eb_spec,
                  eb_spec, full((16, D)), full((D, D)), full((D, D)),
                  full((D, 2 * D)), full((2 * D, D)), full((D, D))],
        out_specs=[pl.BlockSpec((BE, 144), lambda i: (i, 0)), eb_spec],
        out_shape=[jax.ShapeDtypeStruct((EPl, 144), F32),
                   jax.ShapeDtypeStruct((EPl, D), F32)],
        interpret=interpret,
    )(e2d, pp, vs, bond16, we0, woe, we1, we2, wE1)


def _edge1(pp, vs, ep1, E, interpret=False):
    EPl = pp.shape[0]

    def body(p_ref, vs_ref, ep_ref, rows_ref):
        i = pl.program_id(0)
        score = p_ref[...] * ep_ref[...]
        hs = _mm(score, _head_sum_mask())
        rid = lax.broadcasted_iota(jnp.int32, (BE, 1), 0) + i * BE
        msk = (rid < E).astype(F32)
        s = jnp.exp(jnp.clip(hs, -5.0, 5.0)) * msk
        sexp = _mm(s, _head_expand_mask(H))
        rows_ref[...] = jnp.concatenate(
            [vs_ref[...] * sexp, s, jnp.zeros((BE, 8), F32)], axis=1)

    eb_spec = pl.BlockSpec((BE, D), lambda i: (i, 0))
    return pl.pallas_call(
        body, grid=(EPl // BE,),
        in_specs=[eb_spec, eb_spec, eb_spec],
        out_specs=pl.BlockSpec((BE, 144), lambda i: (i, 0)),
        out_shape=jax.ShapeDtypeStruct((EPl, 144), F32),
        interpret=interpret,
    )(pp, vs, ep1)


def _node_block(a_ref, hprev_ref, wo_ref, wh1_ref, wh2_ref):
    a = a_ref[0] + a_ref[1]
    num = a[:, :D]
    den8 = a[:, D:D + 8]
    h_att = num / (_mm(den8, _head_expand_mask(8)) + 1e-6)
    h1 = _ln(hprev_ref[...] + _mm(h_att, wo_ref[...]))
    return _ln(h1 + _mm(jnp.maximum(_mm(h1, wh1_ref[...]), 0.0), wh2_ref[...]))


def _mid(acc3, hv0, wo, wh1, wh2, wq, wk, wv, interpret=False):
    NPl = hv0.shape[0]

    def body(a_ref, hv_ref, wo_ref, wh1_ref, wh2_ref,
             wq_ref, wk_ref, wv_ref, h2_ref, q_ref, k_ref, v_ref):
        h2 = _node_block(a_ref, hv_ref, wo_ref, wh1_ref, wh2_ref)
        h2_ref[...] = h2
        q_ref[...] = _mm(h2, wq_ref[...]) * 0.25
        k_ref[...] = _mm(h2, wk_ref[...])
        v_ref[...] = _mm(h2, wv_ref[...])

    full = lambda shape: pl.BlockSpec(shape, lambda i: (0, 0))
    nb = pl.BlockSpec((BN, D), lambda i: (i, 0))
    return pl.pallas_call(
        body, grid=(NPl // BN,),
        in_specs=[pl.BlockSpec((2, BN, 144), lambda i: (0, i, 0)),
                  nb, full((D, D)), full((D, 2 * D)), full((2 * D, D)),
                  full((D, D)), full((D, D)), full((D, D))],
        out_specs=[nb] * 4,
        out_shape=[jax.ShapeDtypeStruct((NPl, D), F32)] * 4,
        interpret=interpret,
    )(acc3, hv0, wo, wh1, wh2, wq, wk, wv)


def _fin(acc3, h2prev, wo, wh1, wh2, fnh, fnhb, interpret=False):
    NPl = h2prev.shape[0]

    def body(a_ref, hp_ref, wo_ref, wh1_ref, wh2_ref,
             fnh_ref, fb_ref, hsc_ref):
        h2 = _node_block(a_ref, hp_ref, wo_ref, wh1_ref, wh2_ref)
        hsc_ref[...] = _mm(h2, fnh_ref[...]) + fb_ref[0:1, 0:1]

    full = lambda shape: pl.BlockSpec(shape, lambda i: (0, 0))
    nb = pl.BlockSpec((BN, D), lambda i: (i, 0))
    return pl.pallas_call(
        body, grid=(NPl // BN,),
        in_specs=[pl.BlockSpec((2, BN, 144), lambda i: (0, i, 0)),
                  nb, full((D, D)), full((D, 2 * D)), full((2 * D, D)),
                  full((D, 1)), full((1, 1))],
        out_specs=pl.BlockSpec((BN, 1), lambda i: (i, 0)),
        out_shape=jax.ShapeDtypeStruct((NPl, 1), F32),
        interpret=interpret,
    )(acc3, h2prev, wo, wh1, wh2, fnh, fnhb)


def _hist(sb8, sdp, hsc, interpret=False):
    NPl = sdp.shape[0]

    def body(sb_ref, sd_ref, hs_ref, out_ref):
        i = pl.program_id(0)

        @pl.when(i == 0)
        def _init():
            out_ref[...] = jnp.zeros((T, 1), F32)

        start = sb_ref[0:1, 0:1]
        end = sb_ref[1:2, 0:1]
        b = (sd_ref[...] - 1) // 7 + 1
        valid = ((b + 3) >= start) & ((b + 3) <= end)
        hv = hs_ref[...]
        acc = jnp.zeros((T, 1), F32)
        for dtap in range(4):
            pos = b + dtap - start
            m = valid & (pos >= 0) & (pos < T)
            contrib = hv * ((1.0 - 0.25 * dtap) * m.astype(F32))
            posc = jnp.clip(pos, 0, T - 1)
            oh = (lax.broadcasted_iota(jnp.int32, (BN, T), 1) == posc).astype(F32)
            acc = acc + lax.dot_general(oh, contrib, (((0,), (0,)), ((), ())),
                                        preferred_element_type=F32)
        out_ref[...] = out_ref[...] + acc

    full = lambda shape: pl.BlockSpec(shape, lambda i: (0, 0))
    return pl.pallas_call(
        body, grid=(NPl // BN,),
        in_specs=[full((8, 1)), pl.BlockSpec((BN, 1), lambda i: (i, 0)),
                  pl.BlockSpec((BN, 1), lambda i: (i, 0))],
        out_specs=pl.BlockSpec((T, 1), lambda i: (0, 0)),
        out_shape=jax.ShapeDtypeStruct((T, 1), F32),
        interpret=interpret,
    )(sb8, sdp, hsc)


def _mlp(ei, x104, L0, fnxh8, fnxb, w1, b1row, w2, b2, interpret=False):
    def body(ei_ref, x_ref, fw_ref, fb_ref, w1_ref, b1_ref, w2_ref, b2_ref, o_ref):
        eiv = ei_ref[...]
        enh = (x_ref[...] * fw_ref[0:1, 0:1] + eiv * fw_ref[1:2, 0:1]
               + fb_ref[0:1, 0:1])
        idx = lax.broadcasted_iota(jnp.int32, (T, 1), 0)
        final = jnp.where(idx < L0, enh, eiv)
        hid = jnp.maximum(final * w1_ref[...] + b1_ref[...], 0.0)
        o_ref[...] = _mm(hid, w2_ref[...]) + b2_ref[0:1, 0:1]

    full = lambda shape: pl.BlockSpec(shape, lambda i: (0, 0))
    return pl.pallas_call(
        body, grid=(1,),
        in_specs=[full((T, 1)), full((T, 1)), full((8, 1)), full((1, 1)),
                  full((1, 64)), full((1, 64)), full((64, 1)), full((1, 1))],
        out_specs=full((T, 1)),
        out_shape=jax.ShapeDtypeStruct((T, 1), F32),
        interpret=interpret,
    )(ei, x104, fnxh8, fnxb, w1, b1row, w2, b2)


def _sc_gather_pv(k_t, q_t, v_t, src2d, dst2d):
    info = plsc.get_sparse_core_info()
    NC, NS = info.num_cores, info.num_subcores
    NW = NC * NS
    NCH = src2d.shape[0]
    CH = NCH // NW
    G = CH // 2
    Dl = k_t.shape[1]
    EPl = NCH * CHUNK
    mesh = plsc.VectorSubcoreMesh(core_axis_name="c", subcore_axis_name="s")

    @functools.partial(
        pl.kernel,
        out_type=[jax.ShapeDtypeStruct((EPl, Dl), F32)] * 2,
        mesh=mesh,
        scratch_types=[pltpu.VMEM((CH, CHUNK), jnp.int32),
                       pltpu.VMEM((CH, CHUNK), jnp.int32)]
                      + [pltpu.VMEM((CHUNK, Dl), F32)] * 6
                      + [pltpu.SemaphoreType.DMA] * 4,
    )
    def _k(kt, qt, vt, sp, dp, p_o, vs_o, isrc, idst,
           bk0, bq0, bv0, bk1, bq1, bv1, gsem0, gsem1, wsem0, wsem1):
        wid = lax.axis_index("s") * NC + lax.axis_index("c")
        cbase = wid * CH
        pltpu.sync_copy(sp.at[pl.ds(cbase, CH)], isrc)
        pltpu.sync_copy(dp.at[pl.ds(cbase, CH)], idst)
        bufs = ((bk0, bq0, bv0, gsem0, wsem0), (bk1, bq1, bv1, gsem1, wsem1))

        def issue(p, j):
            bk, bq, bv, gsem, _ = bufs[p]
            pltpu.async_copy(kt.at[isrc.at[j]], bk, gsem)
            pltpu.async_copy(qt.at[idst.at[j]], bq, gsem)
            pltpu.async_copy(vt.at[isrc.at[j]], bv, gsem)

        def finish(p, j):
            bk, bq, bv, gsem, wsem = bufs[p]
            for _ in range(3):
                pltpu.make_async_copy(kt.at[isrc.at[0]], bk, gsem).wait()

            def rowfn(r, c):
                for cc in range(Dl // 16):
                    sl = pl.ds(cc * 16, 16)
                    bk[r, sl] = bk[r, sl] * bq[r, sl]
                return c

            lax.fori_loop(0, CHUNK, rowfn, 0)
            base = (cbase + j) * CHUNK
            pltpu.async_copy(bk, p_o.at[pl.ds(base, CHUNK)], wsem)
            pltpu.async_copy(bv, vs_o.at[pl.ds(base, CHUNK)], wsem)

        def drain_wb(p):
            bk, _, _, _, wsem = bufs[p]
            for _ in range(2):
                pltpu.make_async_copy(bk, p_o.at[pl.ds(0, CHUNK)], wsem).wait()

        issue(0, 0)

        def body(g, c):
            jb = 2 * g + 1

            @pl.when(g > 0)
            def _():
                drain_wb(1)

            issue(1, jb)
            finish(0, 2 * g)

            @pl.when(g < G - 1)
            def _():
                drain_wb(0)
                issue(0, 2 * g + 2)

            finish(1, jb)
            return c

        lax.fori_loop(0, G, body, 0)
        drain_wb(0)
        drain_wb(1)

    return _k(k_t, q_t, v_t, src2d, dst2d)


def _sc_scatter(rows, dst2d, zer):
    info = plsc.get_sparse_core_info()
    NC, NS = info.num_cores, info.num_subcores
    NW = NC * NS
    SCH = dst2d.shape[1]
    NCH = dst2d.shape[0]
    CH = NCH // NW
    G = CH // 2
    NPl = zer.shape[0]
    RP = NPl // NS
    mesh = plsc.VectorSubcoreMesh(core_axis_name="c", subcore_axis_name="s")

    @functools.partial(
        pl.kernel,
        out_type=jax.ShapeDtypeStruct((NC * NPl, 144), F32),
        mesh=mesh,
        compiler_params=pltpu.CompilerParams(use_tc_tiling_on_sc=False),
        scratch_types=[pltpu.VMEM((CH, SCH), jnp.int32),
                       pltpu.VMEM((SCH, 144), F32),
                       pltpu.VMEM((SCH, 144), F32),
                       pltpu.VMEM_SHARED((NPl, 144), F32)]
                      + [pltpu.SemaphoreType.DMA] * 4,
    )
    def _k(rw, dp, z, out, idxb, rb0, rb1, acc, fsem0, fsem1, ssem0, ssem1):
        cid = lax.axis_index("c")
        sid = lax.axis_index("s")
        wid = sid * NC + cid
        cbase = wid * CH
        pltpu.sync_copy(dp.at[pl.ds(cbase, CH)], idxb)
        pltpu.sync_copy(z.at[pl.ds(sid * RP, RP)], acc.at[pl.ds(sid * RP, RP)])
        plsc.subcore_barrier()
        bufs = ((rb0, fsem0, ssem0), (rb1, fsem1, ssem1))

        def fetch(p, j):
            rb, fsem, _ = bufs[p]
            pltpu.async_copy(rw.at[pl.ds((cbase + j) * SCH, SCH)], rb, fsem)

        def drain_fetch(p):
            rb, fsem, _ = bufs[p]
            pltpu.make_async_copy(rw.at[pl.ds(0, SCH)], rb, fsem).wait()

        def scat(p, j):
            rb, _, ssem = bufs[p]
            pltpu.async_copy(rb, acc.at[idxb.at[j]], ssem, add=True)

        def drain_scat(p):
            rb, _, ssem = bufs[p]
            pltpu.make_async_copy(rb, acc.at[idxb.at[0]], ssem).wait()

        fetch(0, 0)

        def body(g, c):
            jb = 2 * g + 1

            @pl.when(g > 0)
            def _():
                drain_scat(1)

            fetch(1, jb)
            drain_fetch(0)
            scat(0, 2 * g)

            @pl.when(g < G - 1)
            def _():
                drain_scat(0)
                fetch(0, 2 * g + 2)

            drain_fetch(1)
            scat(1, jb)
            return c

        lax.fori_loop(0, G, body, 0)
        drain_scat(0)
        drain_scat(1)
        plsc.subcore_barrier()
        pltpu.sync_copy(acc.at[pl.ds(sid * RP, RP)],
                        out.at[pl.ds(cid * NPl + sid * RP, RP)])

    return _k(rows, dst2d, zer)


def kernel(x, params, h, e, edge_index, sd, start_bucket, end_bucket):
    p = params
    N = h.shape[0]
    E = e.shape[0]
    NPl = ((N + BN - 1) // BN) * BN
    EPl = ((E + 4095) // 4096) * 4096

    h2d = jnp.pad(h.astype(jnp.int32), (0, NPl - N)).reshape(NPl, 1)
    e2d = jnp.pad(e.astype(jnp.int32), (0, EPl - E)).reshape(EPl, 1)
    srcp = jnp.pad(edge_index[0].astype(jnp.int32), (0, EPl - E))
    dstp = jnp.pad(edge_index[1].astype(jnp.int32), (0, EPl - E))
    sdp = jnp.pad(sd.astype(jnp.int32), (0, NPl - N),
                  constant_values=10 ** 6).reshape(NPl, 1)
    atom128 = jnp.pad(p['atom_emb'], ((0, 128 - p['atom_emb'].shape[0]), (0, 0)))
    bond16 = jnp.pad(p['bond_emb'], ((0, 16 - p['bond_emb'].shape[0]), (0, 0)))
    zer = jnp.zeros((NPl, 144), F32)
    L0 = x.shape[0]
    x104 = jnp.pad(x, ((0, T - L0), (0, 0)))
    sb8 = jnp.zeros((8, 1), jnp.int32)
    sb8 = sb8.at[0, 0].set(jnp.asarray(start_bucket, jnp.int32))
    sb8 = sb8.at[1, 0].set(jnp.asarray(end_bucket, jnp.int32))

    src2d = srcp.reshape(EPl // CHUNK, CHUNK)
    dst2d = dstp.reshape(EPl // CHUNK, CHUNK)
    dst2s = dstp.reshape(EPl // 64, 64)
    hv0, q0, k0, v0 = _prep(h2d, atom128, p['WQ_0'], p['WK_0'], p['WV_0'])
    pp0, vs0 = _sc_gather_pv(k0, q0, v0, src2d, dst2d)
    rows0, ep1 = _edge0(e2d, pp0, vs0, bond16, p['WE_0'], p['WOe_0'],
                        p['We1_0'], p['We2_0'], p['WE_1'], E)
    acc0 = _sc_scatter(rows0, dst2s, zer).reshape(2, NPl, 144)
    h2, q1, k1, v1 = _mid(acc0, hv0, p['WO_0'], p['Wh1_0'], p['Wh2_0'],
                          p['WQ_1'], p['WK_1'], p['WV_1'])
    pp1, vs1 = _sc_gather_pv(k1, q1, v1, src2d, dst2d)
    rows1 = _edge1(pp1, vs1, ep1, E)
    acc1 = _sc_scatter(rows1, dst2s, zer).reshape(2, NPl, 144)
    hsc = _fin(acc1, h2, p['WO_1'], p['Wh1_1'], p['Wh2_1'],
               p['fnh_W'], p['fnh_b'].reshape(1, 1))
    ei = _hist(sb8, sdp, hsc)
    fnxh8 = jnp.pad(p['fnxh_W'].reshape(2, 1), ((0, 6), (0, 0)))
    out = _mlp(ei, x104, L0, fnxh8, p['fnxh_b'].reshape(1, 1),
               p['mlp_W1'], p['mlp_b1'].reshape(1, 64), p['mlp_W2'],
               p['mlp_b2'].reshape(1, 1))
    return out
